# Initial kernel scaffold; baseline (speedup 1.0000x reference)
import jax
import jax.numpy as jnp
from jax.experimental import pallas as pl


def kernel(x, tables):
    raise NotImplementedError("write your pallas kernel here")


# speedup vs baseline: 14.5896x; 14.5896x over previous
import functools

import numpy as np
import jax
import jax.numpy as jnp
from jax import lax
from jax.experimental import pallas as pl
from jax.experimental.pallas import tpu as pltpu
from jax.experimental.pallas import tpu_sc as plsc

NUM_LEVELS = 16
TABLE_SIZE = 2 ** 19
MASK = TABLE_SIZE - 1
FEAT = 2
N = 262144
NC = 2
NS = 16
NW = NC * NS
PER_TILE = N // NW
C = 512
N_CHUNKS = PER_TILE // C
NGROUP = C // 16
GXFER = 128
NXFER = 8 * C // GXFER


def _i32(v: int) -> int:
    v &= 0xFFFFFFFF
    return v - (1 << 32) if v >= (1 << 31) else v


_P1 = _i32(2654435761)
_P2 = _i32(805459861)
_P3 = 3674653429
_SCALES = [float(np.float32(np.float64(1.5) ** l)) for l in range(NUM_LEVELS)]
_KLEV = [_i32(_P3 * l) for l in range(NUM_LEVELS)]


def _hash_encode_sc(x, tab):
    mesh = plsc.VectorSubcoreMesh(core_axis_name="c", subcore_axis_name="s")

    @functools.partial(
        pl.kernel,
        out_type=jax.ShapeDtypeStruct((N, NUM_LEVELS * FEAT), jnp.float32),
        mesh=mesh,
        compiler_params=pltpu.CompilerParams(needs_layout_passes=False,
                                             use_tc_tiling_on_sc=False),
        scratch_types=[
            pltpu.VMEM((C, 3), jnp.float32),
            pltpu.VMEM((C, NUM_LEVELS * FEAT), jnp.float32),
            pltpu.VMEM((NXFER, GXFER), jnp.int32),
            pltpu.VMEM((NXFER, GXFER), jnp.int32),
            pltpu.VMEM((NXFER, GXFER), jnp.int32),
            pltpu.VMEM((NXFER, GXFER), jnp.int32),
            pltpu.VMEM((8 * C,), jnp.float32),
            pltpu.VMEM((8 * C,), jnp.float32),
            pltpu.VMEM((8 * C,), jnp.float32),
            pltpu.VMEM((8 * C,), jnp.float32),
            pltpu.SemaphoreType.DMA,
            pltpu.SemaphoreType.DMA,
        ],
    )
    def body(x_hbm, tab_hbm, out_hbm,
             x_v, out_v, ia0, ib0, ia1, ib1, ra0, rb0, ra1, rb1, sem0, sem1):
        i32c = jnp.int32
        wid = lax.axis_index("s") * i32c(NC) + lax.axis_index("c")
        tile_base = wid * i32c(PER_TILE)
        iota = lax.iota(jnp.int32, 16)
        zero_f = jnp.zeros((16,), jnp.float32)
        one_f = jnp.full((16,), 1.0, jnp.float32)
        idx_bufs = ((ia0, ib0), (ia1, ib1))
        rows_bufs = ((ra0, rb0), (ra1, rb1))
        sems = (sem0, sem1)

        def load_x(g, d):
            ridx = jnp.full((16,), g * i32c(16), jnp.int32) + iota
            return plsc.load_gather(x_v, [ridx, jnp.full((16,), d, jnp.int32)])

        def scaled(g, l):
            sc = jnp.full((16,), _SCALES[l], jnp.float32)
            out = []
            for d in range(3):
                xd = load_x(g, d)
                xd = jnp.minimum(jnp.maximum(xd, zero_f), one_f)
                out.append(xd * sc)
            return out

        def pass_a(l, idx_refs):
            klev = jnp.full((16,), _KLEV[l], jnp.int32)
            p1 = jnp.full((16,), _P1, jnp.int32)
            p2 = jnp.full((16,), _P2, jnp.int32)
            mask = jnp.full((16,), MASK, jnp.int32)
            base_l = jnp.full((16,), l * TABLE_SIZE * FEAT, jnp.int32)
            one_i = jnp.full((16,), 1, jnp.int32)

            def g_body(g, _):
                sx = scaled(g, l)
                xi = [s.astype(jnp.int32) for s in sx]
                a0 = xi[0]
                a1 = xi[0] + one_i
                mm1 = xi[1] * p1
                m1 = (mm1, mm1 + p1)
                mm2 = xi[2] * p2
                m2 = (mm2 ^ klev, (mm2 + p2) ^ klev)
                t = ((a0 ^ m1[0], a0 ^ m1[1]), (a1 ^ m1[0], a1 ^ m1[1]))
                row = g >> i32c(3)
                col = (g & i32c(7)) * i32c(16)
                for c in range(8):
                    b0, b1, b2 = c & 1, (c >> 1) & 1, (c >> 2) & 1
                    h = (t[b0][b1] ^ m2[b2]) & mask
                    e0 = h + h + base_l
                    r = i32c(c * (C // GXFER)) + row
                    idx_refs[0][r, pl.ds(col, 16)] = e0
                    idx_refs[1][r, pl.ds(col, 16)] = e0 + one_i
                return jnp.int32(0)

            lax.fori_loop(jnp.int32(0), jnp.int32(NGROUP), g_body, jnp.int32(0))

        def fire(idx_refs, rows_refs, sem):
            def f_body(j, _):
                off = j * i32c(GXFER)
                for f in range(FEAT):
                    pltpu.async_copy(
                        tab_hbm.at[idx_refs[f].at[j]],
                        rows_refs[f].at[pl.ds(off, GXFER)],
                        sem)
                return jnp.int32(0)

            lax.fori_loop(jnp.int32(0), jnp.int32(NXFER), f_body, jnp.int32(0))

        def drain(idx_refs, rows_refs, sem):
            def d_body(j, _):
                off = j * i32c(GXFER)
                for f in range(FEAT):
                    pltpu.make_async_copy(
                        tab_hbm.at[idx_refs[f].at[j]],
                        rows_refs[f].at[pl.ds(off, GXFER)],
                        sem).wait()
                return jnp.int32(0)

            lax.fori_loop(jnp.int32(0), jnp.int32(NXFER), d_body, jnp.int32(0))

        def pass_b(l, rows_refs):
            def g_body(g, _):
                sx = scaled(g, l)
                xf = [s - s.astype(jnp.int32).astype(jnp.float32) for s in sx]
                w0 = [one_f - f for f in xf]
                wxy = ((w0[0] * w0[1], w0[0] * xf[1]),
                       (xf[0] * w0[1], xf[0] * xf[1]))
                wz = (w0[2], xf[2])
                row0 = jnp.full((16,), g * i32c(16), jnp.int32) + iota
                acc0 = zero_f
                acc1 = zero_f
                for c in range(8):
                    b0, b1, b2 = c & 1, (c >> 1) & 1, (c >> 2) & 1
                    w = wxy[b0][b1] * wz[b2]
                    ridx = row0 + jnp.full((16,), c * C, jnp.int32)
                    f0 = plsc.load_gather(rows_refs[0], [ridx])
                    f1 = plsc.load_gather(rows_refs[1], [ridx])
                    acc0 = acc0 + w * f0
                    acc1 = acc1 + w * f1
                plsc.store_scatter(
                    out_v, [row0, jnp.full((16,), 2 * l, jnp.int32)], acc0)
                plsc.store_scatter(
                    out_v, [row0, jnp.full((16,), 2 * l + 1, jnp.int32)], acc1)
                return jnp.int32(0)

            lax.fori_loop(jnp.int32(0), jnp.int32(NGROUP), g_body, jnp.int32(0))

        def chunk_body(ch, _):
            base = tile_base + ch * i32c(C)
            pltpu.sync_copy(x_hbm.at[pl.ds(base, C), :], x_v)
            pass_a(0, idx_bufs[0])
            fire(idx_bufs[0], rows_bufs[0], sems[0])
            for l in range(NUM_LEVELS):
                b = l % 2
                nb = 1 - b
                if l + 1 < NUM_LEVELS:
                    pass_a(l + 1, idx_bufs[nb])
                    fire(idx_bufs[nb], rows_bufs[nb], sems[nb])
                drain(idx_bufs[b], rows_bufs[b], sems[b])
                pass_b(l, rows_bufs[b])
            pltpu.sync_copy(out_v, out_hbm.at[pl.ds(base, C), :])
            return jnp.int32(0)

        lax.fori_loop(jnp.int32(0), jnp.int32(N_CHUNKS), chunk_body, jnp.int32(0))

    return body(x, tab)


def kernel(x, tables):
    tab = tables.reshape(NUM_LEVELS * TABLE_SIZE * FEAT)
    return _hash_encode_sc(x, tab)
